# Initial kernel scaffold; baseline (speedup 1.0000x reference)
import functools

import jax
import jax.numpy as jnp
from jax import lax
from jax.experimental import pallas as pl
from jax.experimental.pallas import tpu as pltpu
from jax.experimental.pallas import tpu_sc as plsc

N = 10000
E = 320000
D_IN = 128
D_HID = 64
N_CLASSES = 40
ALPHA = 0.2

NC = 2
NS = 16
NW = NC * NS
EP = E // NW
C = 80
NCH = EP // C
ROWS_PT = N // NS


def _dense1_body(x_ref, w_ref, asrc_ref, adst_ref,
                 hext_ref, as_ref, ad_ref, m_ref):
    h = jnp.dot(x_ref[...], w_ref[...], preferred_element_type=jnp.float32)
    asv = jnp.dot(h, asrc_ref[...], preferred_element_type=jnp.float32)
    adv = jnp.dot(h, adst_ref[...], preferred_element_type=jnp.float32)
    ones = jnp.ones((N, 1), jnp.float32)
    pad = jnp.zeros((N, 15), jnp.float32)
    hext_ref[...] = jnp.concatenate([h, ones, pad], axis=1)
    as_ref[...] = asv
    ad_ref[...] = adv
    m = jnp.maximum(jnp.max(asv) + jnp.max(adv), 0.0)
    m_ref[...] = jnp.full((1, 16), m, jnp.float32)


def _dense2_body(acc_ref, b1_ref, w2_ref, asrc_ref, adst_ref,
                 hext_ref, as_ref, ad_ref, m_ref):
    s = acc_ref[0] + acc_ref[1]
    num = s[:, :D_HID]
    den = s[:, D_HID:D_HID + 1]
    x = num / (den + 1e-16) + b1_ref[...]
    x = jnp.where(x > 0, x, jnp.expm1(x))
    h = jnp.dot(x, w2_ref[...], preferred_element_type=jnp.float32)
    asv = jnp.dot(h, asrc_ref[...], preferred_element_type=jnp.float32)
    adv = jnp.dot(h, adst_ref[...], preferred_element_type=jnp.float32)
    ones = jnp.ones((N, 1), jnp.float32)
    pad = jnp.zeros((N, 7), jnp.float32)
    hext_ref[...] = jnp.concatenate([h, ones, pad], axis=1)
    as_ref[...] = asv
    ad_ref[...] = adv
    m = jnp.maximum(jnp.max(asv) + jnp.max(adv), 0.0)
    m_ref[...] = jnp.full((1, 16), m, jnp.float32)


def _final_body(acc_ref, b2_ref, out_ref):
    s = acc_ref[0] + acc_ref[1]
    num = s[:, :N_CLASSES]
    den = s[:, N_CLASSES:N_CLASSES + 1]
    x = num / (den + 1e-16) + b2_ref[...]
    x = jnp.where(x > 0, x, jnp.expm1(x))
    xm = jnp.max(x, axis=1, keepdims=True)
    z = x - xm
    out_ref[...] = z - jnp.log(jnp.sum(jnp.exp(z), axis=1, keepdims=True))


def _make_edge_pass(de):
    mesh = plsc.VectorSubcoreMesh(core_axis_name="c", subcore_axis_name="s")

    @functools.partial(
        pl.kernel,
        out_type=jax.ShapeDtypeStruct((NC, N, de), jnp.float32),
        mesh=mesh,
        scratch_types=[
            pltpu.VMEM_SHARED((N, de), jnp.float32),
            pltpu.VMEM((NCH, C), jnp.int32),
            pltpu.VMEM((NCH, C), jnp.int32),
            pltpu.VMEM((N,), jnp.float32),
            pltpu.VMEM((N,), jnp.float32),
            pltpu.VMEM((16,), jnp.float32),
            pltpu.VMEM((C, de), jnp.float32),
            pltpu.VMEM((C,), jnp.float32),
            pltpu.VMEM((125, de), jnp.float32),
            pltpu.SemaphoreType.DMA,
        ],
    )
    def edge_pass(src_hbm, dst_hbm, hext_hbm, as_hbm, ad_hbm, m_hbm,
                  out_hbm, acc, src2d, dst2d, asv, adv, mv, rows, wv, zb,
                  sem):
        cid = lax.axis_index("c")
        sid = lax.axis_index("s")
        wid = sid * NC + cid

        pltpu.sync_copy(src_hbm.at[pl.ds(wid * NCH, NCH)], src2d)
        pltpu.sync_copy(dst_hbm.at[pl.ds(wid * NCH, NCH)], dst2d)
        pltpu.sync_copy(as_hbm, asv)
        pltpu.sync_copy(ad_hbm, adv)
        pltpu.sync_copy(m_hbm, mv)

        zero = jnp.zeros((16,), jnp.float32)

        def zrow(i, _):
            for j in range(de // 16):
                zb[i, pl.ds(j * 16, 16)] = zero
            return 0

        lax.fori_loop(0, 125, zrow, 0)
        base = sid * ROWS_PT
        for r in range(ROWS_PT // 125):
            pltpu.sync_copy(zb, acc.at[pl.ds(base + r * 125, 125)])
        plsc.subcore_barrier()

        mreg = mv[...]

        def chunk(ci, _):
            pltpu.async_copy(hext_hbm.at[src2d.at[ci]], rows, sem).wait()
            for i in range(C // 16):
                sl = pl.ds(i * 16, 16)
                si = src2d[ci, sl]
                di = dst2d[ci, sl]
                t = plsc.load_gather(asv, [si]) + plsc.load_gather(adv, [di])
                e = jnp.maximum(t, ALPHA * t) - mreg
                wv[sl] = jnp.exp(e)
            for ei in range(C):
                w = wv[ei]
                for j in range(de // 16):
                    sl = pl.ds(j * 16, 16)
                    rows[ei, sl] = rows[ei, sl] * w
            pltpu.sync_copy(rows, acc.at[dst2d.at[ci]], add=True)
            return 0

        lax.fori_loop(0, NCH, chunk, 0)

        plsc.subcore_barrier()
        pltpu.sync_copy(acc.at[pl.ds(base, ROWS_PT)],
                        out_hbm.at[cid, pl.ds(base, ROWS_PT)])

    return edge_pass


_edge_pass_80 = _make_edge_pass(80)
_edge_pass_48 = _make_edge_pass(48)


def kernel(features, edge_index, W1, a1_src, a1_dst, b1,
           W2, a2_src, a2_dst, b2):
    src_r = edge_index[0].reshape(NW * NCH, C)
    dst_r = edge_index[1].reshape(NW * NCH, C)

    h1ext, as1, ad1, m1 = pl.pallas_call(
        _dense1_body,
        out_shape=(
            jax.ShapeDtypeStruct((N, 80), jnp.float32),
            jax.ShapeDtypeStruct((N, 1), jnp.float32),
            jax.ShapeDtypeStruct((N, 1), jnp.float32),
            jax.ShapeDtypeStruct((1, 16), jnp.float32),
        ),
    )(features, W1, a1_src.reshape(D_HID, 1), a1_dst.reshape(D_HID, 1))

    acc1 = _edge_pass_80(src_r, dst_r, h1ext,
                         as1.reshape(N), ad1.reshape(N), m1.reshape(16))

    h2ext, as2, ad2, m2 = pl.pallas_call(
        _dense2_body,
        out_shape=(
            jax.ShapeDtypeStruct((N, 48), jnp.float32),
            jax.ShapeDtypeStruct((N, 1), jnp.float32),
            jax.ShapeDtypeStruct((N, 1), jnp.float32),
            jax.ShapeDtypeStruct((1, 16), jnp.float32),
        ),
    )(acc1, b1.reshape(1, D_HID), W2,
      a2_src.reshape(N_CLASSES, 1), a2_dst.reshape(N_CLASSES, 1))

    acc2 = _edge_pass_48(src_r, dst_r, h2ext,
                         as2.reshape(N), ad2.reshape(N), m2.reshape(16))

    out = pl.pallas_call(
        _final_body,
        out_shape=jax.ShapeDtypeStruct((N, N_CLASSES), jnp.float32),
    )(acc2, b2.reshape(1, N_CLASSES))
    return out


# speedup vs baseline: 41.4450x; 41.4450x over previous
import functools

import jax
import jax.numpy as jnp
from jax import lax
from jax.experimental import pallas as pl
from jax.experimental.pallas import tpu as pltpu
from jax.experimental.pallas import tpu_sc as plsc

N = 10000
E = 320000
D_IN = 128
D_HID = 64
N_CLASSES = 40
ALPHA = 0.2

NC = 2
NS = 16
NW = NC * NS
EP = E // NW
C = 80
NCH = EP // C
ROWS_PT = N // NS


def _dense1_body(x_ref, w_ref, asrc_ref, adst_ref,
                 hext_ref, as_ref, ad_ref, m_ref):
    h = jnp.dot(x_ref[...], w_ref[...], preferred_element_type=jnp.float32)
    asv = jnp.dot(h, asrc_ref[...], preferred_element_type=jnp.float32)
    adv = jnp.dot(h, adst_ref[...], preferred_element_type=jnp.float32)
    ones = jnp.ones((N, 1), jnp.float32)
    pad = jnp.zeros((N, 15), jnp.float32)
    hext_ref[...] = jnp.concatenate([h, ones, pad], axis=1)
    as_ref[...] = asv
    ad_ref[...] = adv
    m = jnp.maximum(jnp.max(asv) + jnp.max(adv), 0.0)
    m_ref[...] = jnp.full((1, 16), m, jnp.float32)


def _dense2_body(acc_ref, b1_ref, w2_ref, asrc_ref, adst_ref,
                 hext_ref, as_ref, ad_ref, m_ref):
    s = acc_ref[0] + acc_ref[1]
    num = s[:, :D_HID]
    den = s[:, D_HID:D_HID + 1]
    x = num / (den + 1e-16) + b1_ref[...]
    x = jnp.where(x > 0, x, jnp.exp(x) - 1.0)
    h = jnp.dot(x, w2_ref[...], preferred_element_type=jnp.float32)
    asv = jnp.dot(h, asrc_ref[...], preferred_element_type=jnp.float32)
    adv = jnp.dot(h, adst_ref[...], preferred_element_type=jnp.float32)
    ones = jnp.ones((N, 1), jnp.float32)
    pad = jnp.zeros((N, 7), jnp.float32)
    hext_ref[...] = jnp.concatenate([h, ones, pad], axis=1)
    as_ref[...] = asv
    ad_ref[...] = adv
    m = jnp.maximum(jnp.max(asv) + jnp.max(adv), 0.0)
    m_ref[...] = jnp.full((1, 16), m, jnp.float32)


def _final_body(acc_ref, b2_ref, out_ref):
    s = acc_ref[0] + acc_ref[1]
    num = s[:, :N_CLASSES]
    den = s[:, N_CLASSES:N_CLASSES + 1]
    x = num / (den + 1e-16) + b2_ref[...]
    x = jnp.where(x > 0, x, jnp.exp(x) - 1.0)
    xm = jnp.max(x, axis=1, keepdims=True)
    z = x - xm
    out_ref[...] = z - jnp.log(jnp.sum(jnp.exp(z), axis=1, keepdims=True))


def _make_edge_pass(de):
    mesh = plsc.VectorSubcoreMesh(core_axis_name="c", subcore_axis_name="s")

    @functools.partial(
        pl.kernel,
        out_type=jax.ShapeDtypeStruct((NC, N, de), jnp.float32),
        mesh=mesh,
        compiler_params=pltpu.CompilerParams(needs_layout_passes=False,
                                             use_tc_tiling_on_sc=False),
        scratch_types=[
            pltpu.VMEM_SHARED((N, de), jnp.float32),
            pltpu.VMEM((NCH, C), jnp.int32),
            pltpu.VMEM((NCH, C), jnp.int32),
            pltpu.VMEM((N,), jnp.float32),
            pltpu.VMEM((N,), jnp.float32),
            pltpu.VMEM((16,), jnp.float32),
            pltpu.VMEM((C, de), jnp.float32),
            pltpu.VMEM((16, de), jnp.float32),
            pltpu.SemaphoreType.DMA,
        ],
    )
    def edge_pass(src_hbm, dst_hbm, hext_hbm, as_hbm, ad_hbm, m_hbm,
                  out_hbm, acc, src2d, dst2d, asv, adv, mv, rows, zb,
                  sem):
        cid = lax.axis_index("c")
        sid = lax.axis_index("s")
        wid = sid * NC + cid

        pltpu.sync_copy(src_hbm.at[wid], src2d)
        pltpu.sync_copy(dst_hbm.at[wid], dst2d)
        pltpu.sync_copy(as_hbm, asv)
        pltpu.sync_copy(ad_hbm, adv)
        pltpu.sync_copy(m_hbm, mv)

        zero = jnp.zeros((16,), jnp.float32)
        for r in range(16):
            for j in range(de // 16):
                zb[r, pl.ds(j * 16, 16)] = zero

        boff = sid * 624

        def zcopy(i, _):
            pltpu.sync_copy(zb, acc.at[pl.ds(boff + i * 16, 16)])
            return 0

        lax.fori_loop(0, 39, zcopy, 0)

        @pl.when(sid == NS - 1)
        def _():
            pltpu.sync_copy(zb, acc.at[pl.ds(N - 16, 16)])

        plsc.subcore_barrier()

        mreg = mv[...]

        def chunk(ci, _):
            pltpu.async_copy(hext_hbm.at[src2d.at[ci]], rows, sem).wait()
            for i in range(C // 16):
                sl = pl.ds(i * 16, 16)
                si = src2d[ci, sl]
                di = dst2d[ci, sl]
                t = plsc.load_gather(asv, [si]) + plsc.load_gather(adv, [di])
                e = jnp.maximum(t, ALPHA * t) - mreg
                wvec = jnp.exp(e)
                for k in range(16):
                    w = wvec[k]
                    ei = i * 16 + k
                    for j in range(de // 16):
                        slj = pl.ds(j * 16, 16)
                        rows[ei, slj] = rows[ei, slj] * w
            pltpu.sync_copy(rows, acc.at[dst2d.at[ci]], add=True)
            return 0

        lax.fori_loop(0, NCH, chunk, 0)

        plsc.subcore_barrier()
        pltpu.sync_copy(acc.at[pl.ds(boff, 624)],
                        out_hbm.at[cid, pl.ds(boff, 624)])

        @pl.when(sid == NS - 1)
        def _():
            pltpu.sync_copy(acc.at[pl.ds(624 * NS, 16)],
                            out_hbm.at[cid, pl.ds(624 * NS, 16)])

    return edge_pass


_edge_pass_80 = _make_edge_pass(80)
_edge_pass_48 = _make_edge_pass(48)


def kernel(features, edge_index, W1, a1_src, a1_dst, b1,
           W2, a2_src, a2_dst, b2):
    src_r = edge_index[0].reshape(NW, NCH, C)
    dst_r = edge_index[1].reshape(NW, NCH, C)

    h1ext, as1, ad1, m1 = pl.pallas_call(
        _dense1_body,
        out_shape=(
            jax.ShapeDtypeStruct((N, 80), jnp.float32),
            jax.ShapeDtypeStruct((N, 1), jnp.float32),
            jax.ShapeDtypeStruct((N, 1), jnp.float32),
            jax.ShapeDtypeStruct((1, 16), jnp.float32),
        ),
    )(features, W1, a1_src.reshape(D_HID, 1), a1_dst.reshape(D_HID, 1))

    acc1 = _edge_pass_80(src_r, dst_r, h1ext,
                         as1.reshape(N), ad1.reshape(N), m1.reshape(16))

    h2ext, as2, ad2, m2 = pl.pallas_call(
        _dense2_body,
        out_shape=(
            jax.ShapeDtypeStruct((N, 48), jnp.float32),
            jax.ShapeDtypeStruct((N, 1), jnp.float32),
            jax.ShapeDtypeStruct((N, 1), jnp.float32),
            jax.ShapeDtypeStruct((1, 16), jnp.float32),
        ),
    )(acc1, b1.reshape(1, D_HID), W2,
      a2_src.reshape(N_CLASSES, 1), a2_dst.reshape(N_CLASSES, 1))

    acc2 = _edge_pass_48(src_r, dst_r, h2ext,
                         as2.reshape(N), ad2.reshape(N), m2.reshape(16))

    out = pl.pallas_call(
        _final_body,
        out_shape=jax.ShapeDtypeStruct((N, N_CLASSES), jnp.float32),
    )(acc2, b2.reshape(1, N_CLASSES))
    return out


# speedup vs baseline: 68.4903x; 1.6526x over previous
import functools

import jax
import jax.numpy as jnp
from jax import lax
from jax.experimental import pallas as pl
from jax.experimental.pallas import tpu as pltpu
from jax.experimental.pallas import tpu_sc as plsc

N = 10000
E = 320000
D_IN = 128
D_HID = 64
N_CLASSES = 40
ALPHA = 0.2

NC = 2
NS = 16
NW = NC * NS
EP = E // NW
C = 80
NCH = EP // C
NB = 5
ROWS_PT = N // NS


def _dense1_body(x_ref, w_ref, asrc_ref, adst_ref,
                 hext_ref, as_ref, ad_ref, m_ref):
    h = jnp.dot(x_ref[...], w_ref[...], preferred_element_type=jnp.float32)
    asv = jnp.dot(h, asrc_ref[...], preferred_element_type=jnp.float32)
    adv = jnp.dot(h, adst_ref[...], preferred_element_type=jnp.float32)
    ones = jnp.ones((N, 1), jnp.float32)
    pad = jnp.zeros((N, 15), jnp.float32)
    hext_ref[...] = jnp.concatenate([h, ones, pad], axis=1)
    as_ref[...] = asv
    ad_ref[...] = adv
    m = jnp.maximum(jnp.max(asv) + jnp.max(adv), 0.0)
    m_ref[...] = jnp.full((1, 16), m, jnp.float32)


def _dense2_body(acc_ref, b1_ref, w2_ref, asrc_ref, adst_ref,
                 hext_ref, as_ref, ad_ref, m_ref):
    s = acc_ref[0] + acc_ref[1]
    num = s[:, :D_HID]
    den = s[:, D_HID:D_HID + 1]
    x = num / (den + 1e-16) + b1_ref[...]
    x = jnp.where(x > 0, x, jnp.exp(x) - 1.0)
    h = jnp.dot(x, w2_ref[...], preferred_element_type=jnp.float32)
    asv = jnp.dot(h, asrc_ref[...], preferred_element_type=jnp.float32)
    adv = jnp.dot(h, adst_ref[...], preferred_element_type=jnp.float32)
    ones = jnp.ones((N, 1), jnp.float32)
    pad = jnp.zeros((N, 7), jnp.float32)
    hext_ref[...] = jnp.concatenate([h, ones, pad], axis=1)
    as_ref[...] = asv
    ad_ref[...] = adv
    m = jnp.maximum(jnp.max(asv) + jnp.max(adv), 0.0)
    m_ref[...] = jnp.full((1, 16), m, jnp.float32)


def _final_body(acc_ref, b2_ref, out_ref):
    s = acc_ref[0] + acc_ref[1]
    num = s[:, :N_CLASSES]
    den = s[:, N_CLASSES:N_CLASSES + 1]
    x = num / (den + 1e-16) + b2_ref[...]
    x = jnp.where(x > 0, x, jnp.exp(x) - 1.0)
    xm = jnp.max(x, axis=1, keepdims=True)
    z = x - xm
    out_ref[...] = z - jnp.log(jnp.sum(jnp.exp(z), axis=1, keepdims=True))


def _make_edge_pass(de):
    mesh = plsc.VectorSubcoreMesh(core_axis_name="c", subcore_axis_name="s")

    @functools.partial(
        pl.kernel,
        out_type=jax.ShapeDtypeStruct((NC, N, de), jnp.float32),
        mesh=mesh,
        compiler_params=pltpu.CompilerParams(needs_layout_passes=False,
                                             use_tc_tiling_on_sc=False),
        scratch_types=[
            pltpu.VMEM_SHARED((N, de), jnp.float32),
            pltpu.VMEM((NCH, C), jnp.int32),
            pltpu.VMEM((NCH, C), jnp.int32),
            pltpu.VMEM((N,), jnp.float32),
            pltpu.VMEM((N,), jnp.float32),
            pltpu.VMEM((16,), jnp.float32),
            pltpu.VMEM((NB, C, de), jnp.float32),
            pltpu.VMEM((16, de), jnp.float32),
            pltpu.SemaphoreType.DMA,
            pltpu.SemaphoreType.DMA,
            pltpu.SemaphoreType.DMA,
            pltpu.SemaphoreType.DMA,
            pltpu.SemaphoreType.DMA,
            pltpu.SemaphoreType.DMA,
            pltpu.SemaphoreType.DMA,
            pltpu.SemaphoreType.DMA,
            pltpu.SemaphoreType.DMA,
            pltpu.SemaphoreType.DMA,
        ],
    )
    def edge_pass(src_hbm, dst_hbm, hext_hbm, as_hbm, ad_hbm, m_hbm,
                  out_hbm, acc, src2d, dst2d, asv, adv, mv, rows, zb,
                  g0, g1, g2, g3, g4, s0, s1, s2, s3, s4):
        gsem = (g0, g1, g2, g3, g4)
        ssem = (s0, s1, s2, s3, s4)
        cid = lax.axis_index("c")
        sid = lax.axis_index("s")
        wid = sid * NC + cid

        pltpu.sync_copy(src_hbm.at[wid], src2d)
        pltpu.sync_copy(dst_hbm.at[wid], dst2d)
        pltpu.sync_copy(as_hbm, asv)
        pltpu.sync_copy(ad_hbm, adv)
        pltpu.sync_copy(m_hbm, mv)

        zero = jnp.zeros((16,), jnp.float32)
        for r in range(16):
            for j in range(de // 16):
                zb[r, pl.ds(j * 16, 16)] = zero

        boff = sid * 624

        def zcopy(i, _):
            pltpu.sync_copy(zb, acc.at[pl.ds(boff + i * 16, 16)])
            return 0

        lax.fori_loop(0, 39, zcopy, 0)

        @pl.when(sid == NS - 1)
        def _():
            pltpu.sync_copy(zb, acc.at[pl.ds(N - 16, 16)])

        plsc.subcore_barrier()

        mreg = mv[...]

        for b in range(NB):
            pltpu.async_copy(hext_hbm.at[src2d.at[b]], rows.at[b], gsem[b])

        def outer(oi, _):
            for b in range(NB):
                ci = oi * NB + b
                bprev = (b - 1) % NB
                wvecs = []
                for i in range(C // 16):
                    sl = pl.ds(i * 16, 16)
                    si = src2d[ci, sl]
                    di = dst2d[ci, sl]
                    t = (plsc.load_gather(asv, [si])
                         + plsc.load_gather(adv, [di]))
                    e = jnp.maximum(t, ALPHA * t) - mreg
                    wvecs.append(jnp.exp(e))
                pltpu.make_async_copy(hext_hbm.at[src2d.at[ci]],
                                      rows.at[b], gsem[b]).wait()
                for i in range(C // 16):
                    for k in range(16):
                        w = wvecs[i][k]
                        ei = i * 16 + k
                        for j in range(de // 16):
                            slj = pl.ds(j * 16, 16)
                            rows[b, ei, slj] = rows[b, ei, slj] * w
                pltpu.async_copy(rows.at[b], acc.at[dst2d.at[ci]],
                                 ssem[b], add=True)
                @pl.when(jnp.logical_and(ci >= 1, ci - 1 + NB < NCH))
                def _():
                    pltpu.make_async_copy(
                        rows.at[bprev], acc.at[dst2d.at[ci - 1]],
                        ssem[bprev]).wait()
                    pltpu.async_copy(hext_hbm.at[src2d.at[ci - 1 + NB]],
                                     rows.at[bprev], gsem[bprev])
            return 0

        lax.fori_loop(0, NCH // NB, outer, 0)
        for t in range(NB):
            ci = NCH - NB + t
            pltpu.make_async_copy(rows.at[ci % NB], acc.at[dst2d.at[ci]],
                                  ssem[ci % NB]).wait()

        plsc.subcore_barrier()
        pltpu.sync_copy(acc.at[pl.ds(boff, 624)],
                        out_hbm.at[cid, pl.ds(boff, 624)])

        @pl.when(sid == NS - 1)
        def _():
            pltpu.sync_copy(acc.at[pl.ds(624 * NS, 16)],
                            out_hbm.at[cid, pl.ds(624 * NS, 16)])

    return edge_pass


_edge_pass_80 = _make_edge_pass(80)
_edge_pass_48 = _make_edge_pass(48)


def kernel(features, edge_index, W1, a1_src, a1_dst, b1,
           W2, a2_src, a2_dst, b2):
    src_r = edge_index[0].reshape(NW, NCH, C)
    dst_r = edge_index[1].reshape(NW, NCH, C)

    h1ext, as1, ad1, m1 = pl.pallas_call(
        _dense1_body,
        out_shape=(
            jax.ShapeDtypeStruct((N, 80), jnp.float32),
            jax.ShapeDtypeStruct((N, 1), jnp.float32),
            jax.ShapeDtypeStruct((N, 1), jnp.float32),
            jax.ShapeDtypeStruct((1, 16), jnp.float32),
        ),
    )(features, W1, a1_src.reshape(D_HID, 1), a1_dst.reshape(D_HID, 1))

    acc1 = _edge_pass_80(src_r, dst_r, h1ext,
                         as1.reshape(N), ad1.reshape(N), m1.reshape(16))

    h2ext, as2, ad2, m2 = pl.pallas_call(
        _dense2_body,
        out_shape=(
            jax.ShapeDtypeStruct((N, 48), jnp.float32),
            jax.ShapeDtypeStruct((N, 1), jnp.float32),
            jax.ShapeDtypeStruct((N, 1), jnp.float32),
            jax.ShapeDtypeStruct((1, 16), jnp.float32),
        ),
    )(acc1, b1.reshape(1, D_HID), W2,
      a2_src.reshape(N_CLASSES, 1), a2_dst.reshape(N_CLASSES, 1))

    acc2 = _edge_pass_48(src_r, dst_r, h2ext,
                         as2.reshape(N), ad2.reshape(N), m2.reshape(16))

    out = pl.pallas_call(
        _final_body,
        out_shape=jax.ShapeDtypeStruct((N, N_CLASSES), jnp.float32),
    )(acc2, b2.reshape(1, N_CLASSES))
    return out


# speedup vs baseline: 70.7144x; 1.0325x over previous
import functools

import jax
import jax.numpy as jnp
from jax import lax
from jax.experimental import pallas as pl
from jax.experimental.pallas import tpu as pltpu
from jax.experimental.pallas import tpu_sc as plsc

N = 10000
E = 320000
D_IN = 128
D_HID = 64
N_CLASSES = 40
ALPHA = 0.2

NC = 2
NS = 16
NW = NC * NS
EP = E // NW
C = 80
NCH = EP // C
NB = 5
ROWS_PT = N // NS


def _dense1_body(x_ref, w_ref, asrc_ref, adst_ref,
                 hext_ref, as_ref, ad_ref, m_ref):
    h = jnp.dot(x_ref[...], w_ref[...], preferred_element_type=jnp.float32)
    asv = jnp.dot(h, asrc_ref[...], preferred_element_type=jnp.float32)
    adv = jnp.dot(h, adst_ref[...], preferred_element_type=jnp.float32)
    ones = jnp.ones((N, 1), jnp.float32)
    pad = jnp.zeros((N, 15), jnp.float32)
    hext_ref[...] = jnp.concatenate([h, ones, pad], axis=1)
    as_ref[...] = asv
    ad_ref[...] = adv
    m = jnp.maximum(jnp.max(asv) + jnp.max(adv), 0.0)
    m_ref[...] = jnp.full((1, 16), m, jnp.float32)


def _dense2_body(acc_ref, b1_ref, w2_ref, asrc_ref, adst_ref,
                 hext_ref, as_ref, ad_ref, m_ref):
    s = acc_ref[0] + acc_ref[1]
    num = s[:, :D_HID]
    den = s[:, D_HID:D_HID + 1]
    x = num / (den + 1e-16) + b1_ref[...]
    x = jnp.where(x > 0, x, jnp.exp(x) - 1.0)
    h = jnp.dot(x, w2_ref[...], preferred_element_type=jnp.float32)
    asv = jnp.dot(h, asrc_ref[...], preferred_element_type=jnp.float32)
    adv = jnp.dot(h, adst_ref[...], preferred_element_type=jnp.float32)
    ones = jnp.ones((N, 1), jnp.float32)
    pad = jnp.zeros((N, 7), jnp.float32)
    hext_ref[...] = jnp.concatenate([h, ones, pad], axis=1)
    as_ref[...] = asv
    ad_ref[...] = adv
    m = jnp.maximum(jnp.max(asv) + jnp.max(adv), 0.0)
    m_ref[...] = jnp.full((1, 16), m, jnp.float32)


def _final_body(acc_ref, b2_ref, out_ref):
    s = acc_ref[0] + acc_ref[1]
    num = s[:, :N_CLASSES]
    den = s[:, N_CLASSES:N_CLASSES + 1]
    x = num / (den + 1e-16) + b2_ref[...]
    x = jnp.where(x > 0, x, jnp.exp(x) - 1.0)
    xm = jnp.max(x, axis=1, keepdims=True)
    z = x - xm
    out_ref[...] = z - jnp.log(jnp.sum(jnp.exp(z), axis=1, keepdims=True))


def _make_edge_pass(de, stage_table):
    mesh = plsc.VectorSubcoreMesh(core_axis_name="c", subcore_axis_name="s")

    @functools.partial(
        pl.kernel,
        out_type=jax.ShapeDtypeStruct((NC, N, de), jnp.float32),
        mesh=mesh,
        compiler_params=pltpu.CompilerParams(needs_layout_passes=False,
                                             use_tc_tiling_on_sc=False),
        scratch_types=[
            pltpu.VMEM_SHARED((N, de), jnp.float32),
            pltpu.VMEM_SHARED((N, de) if stage_table else (8, 16),
                              jnp.float32),
            pltpu.VMEM((NCH, C), jnp.int32),
            pltpu.VMEM((NCH, C), jnp.int32),
            pltpu.VMEM((N,), jnp.float32),
            pltpu.VMEM((N,), jnp.float32),
            pltpu.VMEM((16,), jnp.float32),
            pltpu.VMEM((NB, C, de), jnp.float32),
            pltpu.VMEM((16, de), jnp.float32),
            pltpu.SemaphoreType.DMA,
            pltpu.SemaphoreType.DMA,
            pltpu.SemaphoreType.DMA,
            pltpu.SemaphoreType.DMA,
            pltpu.SemaphoreType.DMA,
            pltpu.SemaphoreType.DMA,
            pltpu.SemaphoreType.DMA,
            pltpu.SemaphoreType.DMA,
            pltpu.SemaphoreType.DMA,
            pltpu.SemaphoreType.DMA,
        ],
    )
    def edge_pass(src_hbm, dst_hbm, hext_hbm, as_hbm, ad_hbm, m_hbm,
                  out_hbm, acc, hspm, src2d, dst2d, asv, adv, mv, rows, zb,
                  g0, g1, g2, g3, g4, s0, s1, s2, s3, s4):
        gsem = (g0, g1, g2, g3, g4)
        ssem = (s0, s1, s2, s3, s4)
        cid = lax.axis_index("c")
        sid = lax.axis_index("s")
        wid = sid * NC + cid

        pltpu.sync_copy(src_hbm.at[wid], src2d)
        pltpu.sync_copy(dst_hbm.at[wid], dst2d)
        pltpu.sync_copy(as_hbm, asv)
        pltpu.sync_copy(ad_hbm, adv)
        pltpu.sync_copy(m_hbm, mv)

        zero = jnp.zeros((16,), jnp.float32)
        for r in range(16):
            for j in range(de // 16):
                zb[r, pl.ds(j * 16, 16)] = zero

        boff = sid * 624
        if stage_table:
            pltpu.sync_copy(hext_hbm.at[pl.ds(boff, 624)],
                            hspm.at[pl.ds(boff, 624)])
        table = hspm if stage_table else hext_hbm

        def zcopy(i, _):
            pltpu.sync_copy(zb, acc.at[pl.ds(boff + i * 16, 16)])
            return 0

        lax.fori_loop(0, 39, zcopy, 0)

        @pl.when(sid == NS - 1)
        def _():
            pltpu.sync_copy(zb, acc.at[pl.ds(N - 16, 16)])
            if stage_table:
                pltpu.sync_copy(hext_hbm.at[pl.ds(N - 16, 16)],
                                hspm.at[pl.ds(N - 16, 16)])

        plsc.subcore_barrier()

        mreg = mv[...]

        for b in range(NB):
            pltpu.async_copy(table.at[src2d.at[b]], rows.at[b], gsem[b])

        def outer(oi, _):
            for b in range(NB):
                ci = oi * NB + b
                bprev = (b - 1) % NB
                wvecs = []
                for i in range(C // 16):
                    sl = pl.ds(i * 16, 16)
                    si = src2d[ci, sl]
                    di = dst2d[ci, sl]
                    t = (plsc.load_gather(asv, [si])
                         + plsc.load_gather(adv, [di]))
                    e = jnp.maximum(t, ALPHA * t) - mreg
                    wvecs.append(jnp.exp(e))
                pltpu.make_async_copy(table.at[src2d.at[ci]],
                                      rows.at[b], gsem[b]).wait()
                for i in range(C // 16):
                    for k in range(16):
                        w = wvecs[i][k]
                        ei = i * 16 + k
                        for j in range(de // 16):
                            slj = pl.ds(j * 16, 16)
                            rows[b, ei, slj] = rows[b, ei, slj] * w
                pltpu.async_copy(rows.at[b], acc.at[dst2d.at[ci]],
                                 ssem[b], add=True)
                @pl.when(jnp.logical_and(ci >= 1, ci - 1 + NB < NCH))
                def _():
                    pltpu.make_async_copy(
                        rows.at[bprev], acc.at[dst2d.at[ci - 1]],
                        ssem[bprev]).wait()
                    pltpu.async_copy(table.at[src2d.at[ci - 1 + NB]],
                                     rows.at[bprev], gsem[bprev])
            return 0

        lax.fori_loop(0, NCH // NB, outer, 0)
        for t in range(NB):
            ci = NCH - NB + t
            pltpu.make_async_copy(rows.at[ci % NB], acc.at[dst2d.at[ci]],
                                  ssem[ci % NB]).wait()

        plsc.subcore_barrier()
        pltpu.sync_copy(acc.at[pl.ds(boff, 624)],
                        out_hbm.at[cid, pl.ds(boff, 624)])

        @pl.when(sid == NS - 1)
        def _():
            pltpu.sync_copy(acc.at[pl.ds(624 * NS, 16)],
                            out_hbm.at[cid, pl.ds(624 * NS, 16)])

    return edge_pass


_edge_pass_80 = _make_edge_pass(80, stage_table=False)
_edge_pass_48 = _make_edge_pass(48, stage_table=True)


def kernel(features, edge_index, W1, a1_src, a1_dst, b1,
           W2, a2_src, a2_dst, b2):
    src_r = edge_index[0].reshape(NW, NCH, C)
    dst_r = edge_index[1].reshape(NW, NCH, C)

    h1ext, as1, ad1, m1 = pl.pallas_call(
        _dense1_body,
        out_shape=(
            jax.ShapeDtypeStruct((N, 80), jnp.float32),
            jax.ShapeDtypeStruct((N, 1), jnp.float32),
            jax.ShapeDtypeStruct((N, 1), jnp.float32),
            jax.ShapeDtypeStruct((1, 16), jnp.float32),
        ),
    )(features, W1, a1_src.reshape(D_HID, 1), a1_dst.reshape(D_HID, 1))

    acc1 = _edge_pass_80(src_r, dst_r, h1ext,
                         as1.reshape(N), ad1.reshape(N), m1.reshape(16))

    h2ext, as2, ad2, m2 = pl.pallas_call(
        _dense2_body,
        out_shape=(
            jax.ShapeDtypeStruct((N, 48), jnp.float32),
            jax.ShapeDtypeStruct((N, 1), jnp.float32),
            jax.ShapeDtypeStruct((N, 1), jnp.float32),
            jax.ShapeDtypeStruct((1, 16), jnp.float32),
        ),
    )(acc1, b1.reshape(1, D_HID), W2,
      a2_src.reshape(N_CLASSES, 1), a2_dst.reshape(N_CLASSES, 1))

    acc2 = _edge_pass_48(src_r, dst_r, h2ext,
                         as2.reshape(N), ad2.reshape(N), m2.reshape(16))

    out = pl.pallas_call(
        _final_body,
        out_shape=jax.ShapeDtypeStruct((N, N_CLASSES), jnp.float32),
    )(acc2, b2.reshape(1, N_CLASSES))
    return out
